# baseline (speedup 1.0000x reference)
import jax
import jax.numpy as jnp
from jax.experimental import pallas as pl

N = 384
NT = 2 * N
D = 128
BI = 32
NBLK = N // BI
REG = 0.001
INVREG = 1.0 / REG
SINK_ITERS = 100
RPAD = 392
CPAD = 512
NEG = -1e30
LOG2E = 1.4426950408889634

_HI = jax.lax.Precision.HIGHEST


def _mm(a, b):
    return jax.lax.dot_general(a, b, (((1,), (0,)), ((), ())),
                               precision=_HI, preferred_element_type=jnp.float32)


def _att_msgs(qs, k, v, kd, vd):
    t = qs[:, None, :] * k[None, :, :]
    mx = jnp.max(t, axis=2, keepdims=True)
    e = jnp.exp2(t - mx)
    z = jnp.sum(e, axis=2, keepdims=True)
    msg = jnp.sum(e * (1.0 / z) * v[None, :, :], axis=1)
    if kd is not None:
        td = qs * kd
        mxd = jnp.max(td, axis=1, keepdims=True)
        ed = jnp.exp2(td - mxd)
        zd = jnp.sum(ed, axis=1, keepdims=True)
        msg = msg - ed * (vd / zd)
    return msg


def _layer(x, w1, b1, w2, b2, w3, b3, q_s, k_s, v_s, o_s, cross):
    q_s[...] = _mm(x, w1) + b1
    k_s[...] = _mm(x, w2) + b2
    v_s[...] = _mm(x, w3) + b3

    if cross:
        k1 = k_s[0:N, :]
        v1 = v_s[0:N, :]

        def blk(i, carry):
            r0 = N + i * BI
            msg = _att_msgs(q_s[pl.ds(r0, BI), :], k1, v1, None, None)
            o_s[pl.ds(r0, BI), :] = msg
            return carry

        jax.lax.fori_loop(0, NBLK, blk, 0)
    else:
        for s in (0, 1):
            ks = k_s[pl.ds(s * N, N), :]
            vs = v_s[pl.ds(s * N, N), :]

            def blk(i, carry):
                r0 = s * N + i * BI
                qs = q_s[pl.ds(r0, BI), :]
                msg = _att_msgs(qs, ks, vs,
                                k_s[pl.ds(r0, BI), :], v_s[pl.ds(r0, BI), :])
                o_s[pl.ds(r0, BI), :] = msg
                return carry

            jax.lax.fori_loop(0, NBLK, blk, 0)


def _mega_kernel(p_ref, d_ref, m_ref,
                 f1w_ref, f1b_ref, f2w_ref, f2b_ref,
                 lw_refs,
                 f3w_ref, f3b_ref, dust_ref,
                 o_ref, q_s, k_s, v_s, o_s):
    p = p_ref[...]
    f1w = f1w_ref[...]
    h = p[:, 0:1] * f1w[0:1, :] + p[:, 1:2] * f1w[1:2, :] + f1b_ref[...]
    h = jnp.maximum(h, 0.0)
    x = jnp.maximum(_mm(h, f2w_ref[...]) + f2b_ref[...], 0.0) + d_ref[...]

    for li, cross in enumerate((False, True, False, True)):
        w1, b1, w2, b2, w3, b3 = (r[...] for r in lw_refs[li])
        _layer(x, w1, b1, w2, b2, w3, b3, q_s, k_s, v_s, o_s, cross)
        msgs = o_s[...]
        if li == 0:
            x = msgs
        elif cross:
            x = jnp.concatenate([x[0:N, :], x[N:, :] + msgs[N:, :]], axis=0)
        else:
            x = x + msgs

    h5 = jnp.maximum(_mm(x, f3w_ref[...]) + f3b_ref[...], 0.0)
    h5 = h5 / jnp.sqrt(jnp.sum(h5 * h5, axis=1, keepdims=True))
    costs = jax.lax.dot_general(h5[0:N, :], h5[N:, :], (((1,), (1,)), ((), ())),
                                precision=_HI,
                                preferred_element_type=jnp.float32)
    w = dust_ref[0, 0]
    mm = 1.0 - costs
    md = 1.0 - w
    logn = jnp.log(float(N))

    def body(_, fg):
        f, g, fd, gd = fg
        td = (gd - md) * INVREG
        xr = (g - mm) * INVREG
        mr = jnp.maximum(jnp.max(xr, axis=1, keepdims=True), td)
        lser = mr + jnp.log(jnp.sum(jnp.exp(xr - mr), axis=1, keepdims=True)
                            + jnp.exp(td - mr))
        f = -REG * lser
        u = (g - md) * INVREG
        mu = jnp.maximum(jnp.max(u, axis=1, keepdims=True), td)
        lsed = mu + jnp.log(jnp.sum(jnp.exp(u - mu), axis=1, keepdims=True)
                            + jnp.exp(td - mu))
        fd = REG * (logn - lsed)

        te = (fd - md) * INVREG
        xc = (f - mm) * INVREG
        mc = jnp.maximum(jnp.max(xc, axis=0, keepdims=True), te)
        lsec = mc + jnp.log(jnp.sum(jnp.exp(xc - mc), axis=0, keepdims=True)
                            + jnp.exp(te - mc))
        g = -REG * lsec
        uc = (f - md) * INVREG
        muc = jnp.maximum(jnp.max(uc, axis=0, keepdims=True), te)
        lsedc = muc + jnp.log(jnp.sum(jnp.exp(uc - muc), axis=0, keepdims=True)
                              + jnp.exp(te - muc))
        gd = REG * (logn - lsedc)
        return f, g, fd, gd

    f0 = jnp.zeros((N, 1), jnp.float32)
    g0 = jnp.zeros((1, N), jnp.float32)
    z0 = jnp.zeros((1, 1), jnp.float32)
    f, g, fd, gd = jax.lax.fori_loop(0, SINK_ITERS, body, (f0, g0, z0, z0))

    sol = jnp.exp((f + g - mm) * INVREG)
    r = m_ref[:, 0:1]
    c = m_ref[:, 1:2]
    rr = jax.lax.broadcasted_iota(jnp.int32, (256, N), 1)
    r_onehot = (rr == r).astype(jnp.float32)
    picked = _mm(r_onehot, sol)
    cc = jax.lax.broadcasted_iota(jnp.int32, (256, N), 1)
    c_onehot = (cc == c).astype(jnp.float32)
    vals = jnp.sum(picked * c_onehot, axis=1, keepdims=True)
    loss = jnp.sum(-jnp.log(vals + 0.001)) * (1.0 / 256.0)
    o_ref[...] = loss.reshape(1, 1)


def _mega_entry(*refs):
    fixed = refs[:7]
    lw = [refs[7 + 6 * i:7 + 6 * (i + 1)] for i in range(4)]
    tail = refs[31:34]
    o_ref = refs[34]
    scratch = refs[35:]
    _mega_kernel(*fixed, lw, *tail, o_ref, *scratch)


def kernel(p1, d1, p2, d2, matches, params):
    from jax.experimental.pallas import tpu as pltpu

    p = jnp.concatenate([p1[0], p2[0]], axis=0)
    d = jnp.concatenate([d1[0], d2[0]], axis=0)
    pr = params
    b = lambda name: pr[name].reshape(1, -1)

    args = [p, d, matches,
            pr['fc1_w'], b('fc1_b'), pr['fc2_w'], b('fc2_b')]
    for l in range(1, 5):
        args += [pr['mp%d_W1' % l] * LOG2E, b('mp%d_b1' % l) * LOG2E,
                 pr['mp%d_W2' % l], b('mp%d_b2' % l),
                 pr['mp%d_W3' % l], b('mp%d_b3' % l)]
    args += [pr['fc3_w'], b('fc3_b'), pr['dustbin'].reshape(1, 1)]

    loss = pl.pallas_call(
        _mega_entry,
        out_shape=jax.ShapeDtypeStruct((1, 1), jnp.float32),
        scratch_shapes=[pltpu.VMEM((NT, D), jnp.float32)] * 4,
    )(*args)
    return loss.reshape(())


# speedup vs baseline: 1.0074x; 1.0074x over previous
import jax
import jax.numpy as jnp
from jax.experimental import pallas as pl

N = 384
NT = 2 * N
D = 128
BI = 32
NBLK = N // BI
REG = 0.001
INVREG = 1.0 / REG
SINK_ITERS = 100
RPAD = 392
CPAD = 512
NEG = -1e30
LOG2E = 1.4426950408889634

_HI = jax.lax.Precision.HIGHEST


def _mm(a, b):
    return jax.lax.dot_general(a, b, (((1,), (0,)), ((), ())),
                               precision=_HI, preferred_element_type=jnp.float32)


def _att_msgs(qs, k, v, kd, vd):
    t = qs[:, None, :] * k[None, :, :]
    mx = jnp.max(t, axis=2, keepdims=True)
    e = jnp.exp2(t - mx)
    z = jnp.sum(e, axis=2, keepdims=True)
    msg = jnp.sum(e * (1.0 / z) * v[None, :, :], axis=1)
    if kd is not None:
        td = qs * kd
        mxd = jnp.max(td, axis=1, keepdims=True)
        ed = jnp.exp2(td - mxd)
        zd = jnp.sum(ed, axis=1, keepdims=True)
        msg = msg - ed * (vd / zd)
    return msg


def _layer(x, w1, b1, w2, b2, w3, b3, q_s, k_s, v_s, o_s, cross):
    q_s[...] = _mm(x, w1) + b1
    k_s[...] = _mm(x, w2) + b2
    v_s[...] = _mm(x, w3) + b3

    if cross:
        k1 = k_s[0:N, :]
        v1 = v_s[0:N, :]

        def blk(i, carry):
            r0 = N + i * BI
            msg = _att_msgs(q_s[pl.ds(r0, BI), :], k1, v1, None, None)
            o_s[pl.ds(r0, BI), :] = msg
            return carry

        jax.lax.fori_loop(0, NBLK, blk, 0)
    else:
        for s in (0, 1):
            ks = k_s[pl.ds(s * N, N), :]
            vs = v_s[pl.ds(s * N, N), :]

            def blk(i, carry):
                r0 = s * N + i * BI
                qs = q_s[pl.ds(r0, BI), :]
                msg = _att_msgs(qs, ks, vs,
                                k_s[pl.ds(r0, BI), :], v_s[pl.ds(r0, BI), :])
                o_s[pl.ds(r0, BI), :] = msg
                return carry

            jax.lax.fori_loop(0, NBLK, blk, 0)


def _mega_kernel(p_ref, d_ref, m_ref,
                 f1w_ref, f1b_ref, f2w_ref, f2b_ref,
                 lw_refs,
                 f3w_ref, f3b_ref, dust_ref,
                 o_ref, q_s, k_s, v_s, o_s):
    p = p_ref[...]
    f1w = f1w_ref[...]
    h = p[:, 0:1] * f1w[0:1, :] + p[:, 1:2] * f1w[1:2, :] + f1b_ref[...]
    h = jnp.maximum(h, 0.0)
    x = jnp.maximum(_mm(h, f2w_ref[...]) + f2b_ref[...], 0.0) + d_ref[...]

    for li, cross in enumerate((False, True, False, True)):
        w1, b1, w2, b2, w3, b3 = (r[...] for r in lw_refs[li])
        _layer(x, w1, b1, w2, b2, w3, b3, q_s, k_s, v_s, o_s, cross)
        msgs = o_s[...]
        if li == 0:
            x = msgs
        elif cross:
            x = jnp.concatenate([x[0:N, :], x[N:, :] + msgs[N:, :]], axis=0)
        else:
            x = x + msgs

    h5 = jnp.maximum(_mm(x, f3w_ref[...]) + f3b_ref[...], 0.0)
    h5 = h5 / jnp.sqrt(jnp.sum(h5 * h5, axis=1, keepdims=True))
    costs = jax.lax.dot_general(h5[0:N, :], h5[N:, :], (((1,), (1,)), ((), ())),
                                precision=_HI,
                                preferred_element_type=jnp.float32)
    w = dust_ref[0, 0]
    cp = jnp.pad(costs, ((0, RPAD - N), (0, CPAD - N)))
    ri = jax.lax.broadcasted_iota(jnp.int32, (RPAD, CPAD), 0)
    ci = jax.lax.broadcasted_iota(jnp.int32, (RPAD, CPAD), 1)
    interior = (ri < N) & (ci < N)
    boundary = (ri <= N) & (ci <= N) & ~interior
    m_mat = jnp.where(interior, 1.0 - cp, jnp.where(boundary, 1.0 - w, 0.0))

    rv = jax.lax.broadcasted_iota(jnp.int32, (RPAD, 1), 0)
    cv = jax.lax.broadcasted_iota(jnp.int32, (1, CPAD), 1)
    row_valid = rv <= N
    col_valid = cv <= N
    loga = jnp.where(rv == N, jnp.log(float(N)), 0.0)
    logb = jnp.where(cv == N, jnp.log(float(N)), 0.0)

    def body(_, fg):
        f, g = fg
        xr = jnp.where(col_valid, (g - m_mat) * INVREG, NEG)
        mr = jnp.max(xr, axis=1, keepdims=True)
        lser = mr + jnp.log(jnp.sum(jnp.exp(xr - mr), axis=1, keepdims=True))
        f = jnp.where(row_valid, REG * (loga - lser), 0.0)
        xc = jnp.where(row_valid, (f - m_mat) * INVREG, NEG)
        mc = jnp.max(xc, axis=0, keepdims=True)
        lsec = mc + jnp.log(jnp.sum(jnp.exp(xc - mc), axis=0, keepdims=True))
        g = jnp.where(col_valid, REG * (logb - lsec), 0.0)
        return f, g

    f0 = jnp.zeros((RPAD, 1), jnp.float32)
    g0 = jnp.zeros((1, CPAD), jnp.float32)
    f, g = jax.lax.fori_loop(0, SINK_ITERS, body, (f0, g0))

    sol = jnp.where((rv < N) & (cv < N),
                    jnp.exp((f + g - m_mat) * INVREG), 0.0)
    r = m_ref[:, 0:1]
    c = m_ref[:, 1:2]
    rr = jax.lax.broadcasted_iota(jnp.int32, (256, RPAD), 1)
    r_onehot = (rr == r).astype(jnp.float32)
    picked = _mm(r_onehot, sol)
    cc = jax.lax.broadcasted_iota(jnp.int32, (256, CPAD), 1)
    c_onehot = (cc == c).astype(jnp.float32)
    vals = jnp.sum(picked * c_onehot, axis=1, keepdims=True)
    loss = jnp.sum(-jnp.log(vals + 0.001)) * (1.0 / 256.0)
    o_ref[...] = loss.reshape(1, 1)


def _mega_entry(*refs):
    fixed = refs[:7]
    lw = [refs[7 + 6 * i:7 + 6 * (i + 1)] for i in range(4)]
    tail = refs[31:34]
    o_ref = refs[34]
    scratch = refs[35:]
    _mega_kernel(*fixed, lw, *tail, o_ref, *scratch)


def kernel(p1, d1, p2, d2, matches, params):
    from jax.experimental.pallas import tpu as pltpu

    p = jnp.concatenate([p1[0], p2[0]], axis=0)
    d = jnp.concatenate([d1[0], d2[0]], axis=0)
    pr = params
    b = lambda name: pr[name].reshape(1, -1)

    args = [p, d, matches,
            pr['fc1_w'], b('fc1_b'), pr['fc2_w'], b('fc2_b')]
    for l in range(1, 5):
        args += [pr['mp%d_W1' % l] * LOG2E, b('mp%d_b1' % l) * LOG2E,
                 pr['mp%d_W2' % l], b('mp%d_b2' % l),
                 pr['mp%d_W3' % l], b('mp%d_b3' % l)]
    args += [pr['fc3_w'], b('fc3_b'), pr['dustbin'].reshape(1, 1)]

    loss = pl.pallas_call(
        _mega_entry,
        out_shape=jax.ShapeDtypeStruct((1, 1), jnp.float32),
        scratch_shapes=[pltpu.VMEM((NT, D), jnp.float32)] * 4,
    )(*args)
    return loss.reshape(())


# speedup vs baseline: 1.0256x; 1.0181x over previous
import jax
import jax.numpy as jnp
from jax.experimental import pallas as pl

N = 384
NT = 2 * N
D = 128
BI = 32
NBLK = N // BI
REG = 0.001
INVREG = 1.0 / REG
SINK_ITERS = 100
RPAD = 392
CPAD = 512
NEG = -1e30
LOG2E = 1.4426950408889634

_HI = jax.lax.Precision.HIGHEST


def _mm(a, b):
    return jax.lax.dot_general(a, b, (((1,), (0,)), ((), ())),
                               precision=_HI, preferred_element_type=jnp.float32)


def _att_msgs(qs, k, v, kd, vd):
    t = qs[:, None, :] * k[None, :, :]
    mx = jnp.max(t, axis=2, keepdims=True)
    e = jnp.exp2(t - mx)
    z = jnp.sum(e, axis=2, keepdims=True)
    msg = jnp.sum(e * (1.0 / z) * v[None, :, :], axis=1)
    if kd is not None:
        td = qs * kd
        mxd = jnp.max(td, axis=1, keepdims=True)
        ed = jnp.exp2(td - mxd)
        zd = jnp.sum(ed, axis=1, keepdims=True)
        msg = msg - ed * (vd / zd)
    return msg


def _layer(x, w1, b1, w2, b2, w3, b3, q_s, k_s, v_s, o_s, cross):
    q_s[...] = (_mm(x, w1) + b1) * LOG2E
    k_s[...] = _mm(x, w2) + b2
    v_s[...] = _mm(x, w3) + b3

    if cross:
        k1 = k_s[0:N, :]
        v1 = v_s[0:N, :]

        def blk(i, carry):
            r0 = N + i * BI
            msg = _att_msgs(q_s[pl.ds(r0, BI), :], k1, v1, None, None)
            o_s[pl.ds(r0, BI), :] = msg
            return carry

        jax.lax.fori_loop(0, NBLK, blk, 0)
    else:
        for s in (0, 1):
            ks = k_s[pl.ds(s * N, N), :]
            vs = v_s[pl.ds(s * N, N), :]

            def blk(i, carry):
                r0 = s * N + i * BI
                qs = q_s[pl.ds(r0, BI), :]
                msg = _att_msgs(qs, ks, vs,
                                k_s[pl.ds(r0, BI), :], v_s[pl.ds(r0, BI), :])
                o_s[pl.ds(r0, BI), :] = msg
                return carry

            jax.lax.fori_loop(0, NBLK, blk, 0)


def _mega_kernel(p_ref, d_ref, m_ref,
                 f1w_ref, f1b_ref, f2w_ref, f2b_ref,
                 lw_refs,
                 f3w_ref, f3b_ref, dust_ref,
                 o_ref, q_s, k_s, v_s, o_s):
    p = p_ref[...]
    f1w = f1w_ref[...]
    h = p[:, 0:1] * f1w[0:1, :] + p[:, 1:2] * f1w[1:2, :] + f1b_ref[...]
    h = jnp.maximum(h, 0.0)
    x = jnp.maximum(_mm(h, f2w_ref[...]) + f2b_ref[...], 0.0) + d_ref[...]

    for li, cross in enumerate((False, True, False, True)):
        w1, b1, w2, b2, w3, b3 = (r[...] for r in lw_refs[li])
        _layer(x, w1, b1, w2, b2, w3, b3, q_s, k_s, v_s, o_s, cross)
        msgs = o_s[...]
        if li == 0:
            x = msgs
        elif cross:
            x = jnp.concatenate([x[0:N, :], x[N:, :] + msgs[N:, :]], axis=0)
        else:
            x = x + msgs

    h5 = jnp.maximum(_mm(x, f3w_ref[...]) + f3b_ref[...], 0.0)
    h5 = h5 / jnp.sqrt(jnp.sum(h5 * h5, axis=1, keepdims=True))
    costs = jax.lax.dot_general(h5[0:N, :], h5[N:, :], (((1,), (1,)), ((), ())),
                                precision=_HI,
                                preferred_element_type=jnp.float32)
    w = dust_ref[0, 0]
    cp = jnp.pad(costs, ((0, RPAD - N), (0, CPAD - N)))
    ri = jax.lax.broadcasted_iota(jnp.int32, (RPAD, CPAD), 0)
    ci = jax.lax.broadcasted_iota(jnp.int32, (RPAD, CPAD), 1)
    interior = (ri < N) & (ci < N)
    boundary = (ri <= N) & (ci <= N) & ~interior
    m_mat = jnp.where(interior, 1.0 - cp, jnp.where(boundary, 1.0 - w, 0.0))

    rv = jax.lax.broadcasted_iota(jnp.int32, (RPAD, 1), 0)
    cv = jax.lax.broadcasted_iota(jnp.int32, (1, CPAD), 1)
    row_valid = rv <= N
    col_valid = cv <= N
    loga = jnp.where(rv == N, jnp.log(float(N)), 0.0)
    logb = jnp.where(cv == N, jnp.log(float(N)), 0.0)

    def body(_, fg):
        f, g = fg
        xr = jnp.where(col_valid, (g - m_mat) * INVREG, NEG)
        mr = jnp.max(xr, axis=1, keepdims=True)
        lser = mr + jnp.log(jnp.sum(jnp.exp(xr - mr), axis=1, keepdims=True))
        f = jnp.where(row_valid, REG * (loga - lser), 0.0)
        xc = jnp.where(row_valid, (f - m_mat) * INVREG, NEG)
        mc = jnp.max(xc, axis=0, keepdims=True)
        lsec = mc + jnp.log(jnp.sum(jnp.exp(xc - mc), axis=0, keepdims=True))
        g = jnp.where(col_valid, REG * (logb - lsec), 0.0)
        return f, g

    f0 = jnp.zeros((RPAD, 1), jnp.float32)
    g0 = jnp.zeros((1, CPAD), jnp.float32)
    f, g = jax.lax.fori_loop(0, SINK_ITERS, body, (f0, g0))

    sol = jnp.where((rv < N) & (cv < N),
                    jnp.exp((f + g - m_mat) * INVREG), 0.0)
    r = m_ref[:, 0:1]
    c = m_ref[:, 1:2]
    rr = jax.lax.broadcasted_iota(jnp.int32, (256, RPAD), 1)
    r_onehot = (rr == r).astype(jnp.float32)
    picked = _mm(r_onehot, sol)
    cc = jax.lax.broadcasted_iota(jnp.int32, (256, CPAD), 1)
    c_onehot = (cc == c).astype(jnp.float32)
    vals = jnp.sum(picked * c_onehot, axis=1, keepdims=True)
    loss = jnp.sum(-jnp.log(vals + 0.001)) * (1.0 / 256.0)
    o_ref[...] = loss.reshape(1, 1)


def _mega_entry(*refs):
    fixed = refs[:7]
    lw = [refs[7 + 6 * i:7 + 6 * (i + 1)] for i in range(4)]
    tail = refs[31:34]
    o_ref = refs[34]
    scratch = refs[35:]
    _mega_kernel(*fixed, lw, *tail, o_ref, *scratch)


def kernel(p1, d1, p2, d2, matches, params):
    from jax.experimental.pallas import tpu as pltpu

    p = jnp.concatenate([p1[0], p2[0]], axis=0)
    d = jnp.concatenate([d1[0], d2[0]], axis=0)
    pr = params
    b = lambda name: pr[name].reshape(1, -1)

    args = [p, d, matches,
            pr['fc1_w'], b('fc1_b'), pr['fc2_w'], b('fc2_b')]
    for l in range(1, 5):
        args += [pr['mp%d_W1' % l], b('mp%d_b1' % l),
                 pr['mp%d_W2' % l], b('mp%d_b2' % l),
                 pr['mp%d_W3' % l], b('mp%d_b3' % l)]
    args += [pr['fc3_w'], b('fc3_b'), pr['dustbin'].reshape(1, 1)]

    loss = pl.pallas_call(
        _mega_entry,
        out_shape=jax.ShapeDtypeStruct((1, 1), jnp.float32),
        scratch_shapes=[pltpu.VMEM((NT, D), jnp.float32)] * 4,
    )(*args)
    return loss.reshape(())


# speedup vs baseline: 1.0272x; 1.0015x over previous
import jax
import jax.numpy as jnp
from jax.experimental import pallas as pl

N = 384
NT = 2 * N
D = 128
BI = 32
NBLK = N // BI
REG = 0.001
INVREG = 1.0 / REG
SINK_ITERS = 100
RPAD = 392
CPAD = 512
NEG = -1e30
LOG2E = 1.4426950408889634

_HI = jax.lax.Precision.HIGHEST


def _mm(a, b):
    return jax.lax.dot_general(a, b, (((1,), (0,)), ((), ())),
                               precision=_HI, preferred_element_type=jnp.float32)


def _att_msgs(qs, k, v):
    t = qs[:, None, :] * k[None, :, :]
    mx = jnp.max(t, axis=2, keepdims=True)
    e = jnp.exp2(t - mx)
    z = jnp.sum(e, axis=2, keepdims=True)
    return jnp.sum(e * (1.0 / z) * v[None, :, :], axis=1)


def _layer(x, w1, b1, w2, b2, w3, b3, q_s, k_s, v_s, o_s, cross):
    q = (_mm(x, w1) + b1) * LOG2E
    k = _mm(x, w2) + b2
    v = _mm(x, w3) + b3
    q_s[...] = q
    k_s[...] = k
    v_s[...] = v

    if cross:
        k1 = k_s[0:N, :]
        v1 = v_s[0:N, :]

        def blk(i, carry):
            r0 = N + i * BI
            o_s[pl.ds(r0, BI), :] = _att_msgs(q_s[pl.ds(r0, BI), :], k1, v1)
            return carry

        jax.lax.fori_loop(0, NBLK, blk, 0)
        return None
    else:
        for s in (0, 1):
            ks = k_s[pl.ds(s * N, N), :]
            vs = v_s[pl.ds(s * N, N), :]

            def blk(i, carry):
                r0 = s * N + i * BI
                o_s[pl.ds(r0, BI), :] = _att_msgs(q_s[pl.ds(r0, BI), :], ks, vs)
                return carry

            jax.lax.fori_loop(0, NBLK, blk, 0)
        td = q * k
        mxd = jnp.max(td, axis=1, keepdims=True)
        ed = jnp.exp2(td - mxd)
        zd = jnp.sum(ed, axis=1, keepdims=True)
        return ed * (v / zd)


def _mega_kernel(p_ref, d_ref, m_ref,
                 f1w_ref, f1b_ref, f2w_ref, f2b_ref,
                 lw_refs,
                 f3w_ref, f3b_ref, dust_ref,
                 o_ref, q_s, k_s, v_s, o_s):
    p = p_ref[...]
    f1w = f1w_ref[...]
    h = p[:, 0:1] * f1w[0:1, :] + p[:, 1:2] * f1w[1:2, :] + f1b_ref[...]
    h = jnp.maximum(h, 0.0)
    x = jnp.maximum(_mm(h, f2w_ref[...]) + f2b_ref[...], 0.0) + d_ref[...]

    for li, cross in enumerate((False, True, False, True)):
        w1, b1, w2, b2, w3, b3 = (r[...] for r in lw_refs[li])
        diag = _layer(x, w1, b1, w2, b2, w3, b3, q_s, k_s, v_s, o_s, cross)
        msgs = o_s[...]
        if li == 0:
            x = msgs - diag
        elif cross:
            x = jnp.concatenate([x[0:N, :], x[N:, :] + msgs[N:, :]], axis=0)
        else:
            x = x + msgs - diag

    h5 = jnp.maximum(_mm(x, f3w_ref[...]) + f3b_ref[...], 0.0)
    h5 = h5 / jnp.sqrt(jnp.sum(h5 * h5, axis=1, keepdims=True))
    costs = jax.lax.dot_general(h5[0:N, :], h5[N:, :], (((1,), (1,)), ((), ())),
                                precision=_HI,
                                preferred_element_type=jnp.float32)
    w = dust_ref[0, 0]
    cp = jnp.pad(costs, ((0, RPAD - N), (0, CPAD - N)))
    ri = jax.lax.broadcasted_iota(jnp.int32, (RPAD, CPAD), 0)
    ci = jax.lax.broadcasted_iota(jnp.int32, (RPAD, CPAD), 1)
    interior = (ri < N) & (ci < N)
    boundary = (ri <= N) & (ci <= N) & ~interior
    m_mat = jnp.where(interior, 1.0 - cp, jnp.where(boundary, 1.0 - w, 0.0))

    rv = jax.lax.broadcasted_iota(jnp.int32, (RPAD, 1), 0)
    cv = jax.lax.broadcasted_iota(jnp.int32, (1, CPAD), 1)
    row_valid = rv <= N
    col_valid = cv <= N
    loga = jnp.where(rv == N, jnp.log(float(N)), 0.0)
    logb = jnp.where(cv == N, jnp.log(float(N)), 0.0)

    def body(_, fg):
        f, g = fg
        xr = jnp.where(col_valid, (g - m_mat) * INVREG, NEG)
        mr = jnp.max(xr, axis=1, keepdims=True)
        lser = mr + jnp.log(jnp.sum(jnp.exp(xr - mr), axis=1, keepdims=True))
        f = jnp.where(row_valid, REG * (loga - lser), 0.0)
        xc = jnp.where(row_valid, (f - m_mat) * INVREG, NEG)
        mc = jnp.max(xc, axis=0, keepdims=True)
        lsec = mc + jnp.log(jnp.sum(jnp.exp(xc - mc), axis=0, keepdims=True))
        g = jnp.where(col_valid, REG * (logb - lsec), 0.0)
        return f, g

    f0 = jnp.zeros((RPAD, 1), jnp.float32)
    g0 = jnp.zeros((1, CPAD), jnp.float32)
    f, g = jax.lax.fori_loop(0, SINK_ITERS, body, (f0, g0))

    sol = jnp.where((rv < N) & (cv < N),
                    jnp.exp((f + g - m_mat) * INVREG), 0.0)
    r = m_ref[:, 0:1]
    c = m_ref[:, 1:2]
    rr = jax.lax.broadcasted_iota(jnp.int32, (256, RPAD), 1)
    r_onehot = (rr == r).astype(jnp.float32)
    picked = _mm(r_onehot, sol)
    cc = jax.lax.broadcasted_iota(jnp.int32, (256, CPAD), 1)
    c_onehot = (cc == c).astype(jnp.float32)
    vals = jnp.sum(picked * c_onehot, axis=1, keepdims=True)
    loss = jnp.sum(-jnp.log(vals + 0.001)) * (1.0 / 256.0)
    o_ref[...] = loss.reshape(1, 1)


def _mega_entry(*refs):
    fixed = refs[:7]
    lw = [refs[7 + 6 * i:7 + 6 * (i + 1)] for i in range(4)]
    tail = refs[31:34]
    o_ref = refs[34]
    scratch = refs[35:]
    _mega_kernel(*fixed, lw, *tail, o_ref, *scratch)


def kernel(p1, d1, p2, d2, matches, params):
    from jax.experimental.pallas import tpu as pltpu

    p = jnp.concatenate([p1[0], p2[0]], axis=0)
    d = jnp.concatenate([d1[0], d2[0]], axis=0)
    pr = params
    b = lambda name: pr[name].reshape(1, -1)

    args = [p, d, matches,
            pr['fc1_w'], b('fc1_b'), pr['fc2_w'], b('fc2_b')]
    for l in range(1, 5):
        args += [pr['mp%d_W1' % l], b('mp%d_b1' % l),
                 pr['mp%d_W2' % l], b('mp%d_b2' % l),
                 pr['mp%d_W3' % l], b('mp%d_b3' % l)]
    args += [pr['fc3_w'], b('fc3_b'), pr['dustbin'].reshape(1, 1)]

    loss = pl.pallas_call(
        _mega_entry,
        out_shape=jax.ShapeDtypeStruct((1, 1), jnp.float32),
        scratch_shapes=[pltpu.VMEM((NT, D), jnp.float32)] * 4,
    )(*args)
    return loss.reshape(())
